# Initial kernel scaffold; baseline (speedup 1.0000x reference)
import jax
import jax.numpy as jnp
from jax.experimental import pallas as pl


def kernel(xyz, features, geo_w1, geo_b1, geo_g1, geo_bt1, geo_w2, geo_b2, ep_w1, ep_b1, ep_g1, ep_bt1, ep_w2, ep_b2, er_w, er_b, er_g, er_bt):
    raise NotImplementedError("write your pallas kernel here")


# speedup vs baseline: 21.7079x; 21.7079x over previous
import functools

import jax
import jax.numpy as jnp
from jax.experimental import pallas as pl
from jax.experimental.pallas import tpu as pltpu

B, N, C, K = 4, 2048, 512, 16
R = 256

_NEG_INF = float("-inf")


def _ln(x, g, b):
    m = x.mean(-1, keepdims=True)
    v = ((x - m) ** 2).mean(-1, keepdims=True)
    return (x - m) / jnp.sqrt(v + 1e-5) * g + b


def _body(xr_ref, xaT_ref, xcat_ref, f_ref,
          gw1_ref, gb1_ref, gg1_ref, gbt1_ref, gw2_ref, gb2_ref,
          epw1a_ref, epw1b_ref, epb1_ref, epg1_ref, epbt1_ref,
          epw2r_ref, epb2_ref,
          erw_ref, erb_ref, erg_ref, erbt_ref,
          refined_ref, edge_ref):
    xr = xr_ref[0]
    xaT = xaT_ref[0]
    xcat = xcat_ref[0]
    f = f_ref[0]

    s = jax.lax.dot_general(xr, xaT, (((1,), (0,)), ((), ())),
                            preferred_element_type=jnp.float32)
    xx_r = jnp.sum(xr * xr, axis=1, keepdims=True)
    xa2T = xaT * xaT
    xx_a = jnp.sum(xa2T, axis=0, keepdims=True)
    pd = (2.0 * s - xx_r) - xx_a

    work = pd
    th = None
    for t in range(K):
        th = jnp.max(work, axis=1, keepdims=True)
        if t < K - 1:
            work = jnp.where(work == th, _NEG_INF, work)
    m = (pd >= th).astype(jnp.float32)

    s12 = jax.lax.dot_general(m, xcat, (((1,), (0,)), ((), ())),
                              preferred_element_type=jnp.float32)
    cen = s12[:, 0:8] * (1.0 / K)
    ssq = s12[:, 8:16]
    var = (ssq - K * cen * cen) * (1.0 / K)
    vx, vy, vz = var[:, 0:1], var[:, 1:2], var[:, 2:3]
    tr = vx + vy + vz
    det = jnp.clip(vx * vy * vz, 1e-8, None)
    kk = jnp.float32(K) / jnp.float32(K - 1)
    sx = jnp.sqrt(jnp.maximum(vx * kk, 0.0))
    sy = jnp.sqrt(jnp.maximum(vy * kk, 0.0))
    sz = jnp.sqrt(jnp.maximum(vz * kk, 0.0))

    scen = jax.lax.dot_general(cen, xaT, (((1,), (0,)), ((), ())),
                               preferred_element_type=jnp.float32)
    cc = jnp.sum(cen * cen, axis=1, keepdims=True)
    d2 = jnp.maximum((cc - 2.0 * scen) + xx_a, 0.0)
    rtd = jnp.sqrt(d2)
    s1 = jnp.sum(m * rtd, axis=1, keepdims=True)
    s2 = jnp.sum(m * d2, axis=1, keepdims=True)
    md = s1 * (1.0 / K)
    sd = jnp.sqrt(jnp.maximum(s2 - K * md * md, 0.0) * (1.0 / (K - 1)))

    tr6 = tr + 1e-6
    geo = jnp.concatenate(
        [tr, det, sx, sy, sz, md, sd, sx / tr6, sy / tr6, sz / tr6,
         jnp.zeros((xr.shape[0], 6), jnp.float32)], axis=1)

    h1 = jax.lax.dot_general(geo, gw1_ref[...], (((1,), (0,)), ((), ())),
                             preferred_element_type=jnp.float32) + gb1_ref[...]
    h1 = jnp.maximum(_ln(h1, gg1_ref[...], gbt1_ref[...]), 0.0)
    h2 = jax.lax.dot_general(h1, gw2_ref[...], (((1,), (0,)), ((), ())),
                             preferred_element_type=jnp.float32) + gb2_ref[...]

    e1 = (jax.lax.dot_general(f, epw1a_ref[...], (((1,), (0,)), ((), ())),
                              preferred_element_type=jnp.float32)
          + jax.lax.dot_general(h2, epw1b_ref[...], (((1,), (0,)), ((), ())),
                                preferred_element_type=jnp.float32)
          + epb1_ref[...])
    e1 = jnp.maximum(_ln(e1, epg1_ref[...], epbt1_ref[...]), 0.0)
    e2 = jnp.sum(e1 * epw2r_ref[...], axis=1, keepdims=True) + epb2_ref[...]
    edge = jax.nn.sigmoid(e2)

    r1 = jax.lax.dot_general(f, erw_ref[...], (((1,), (0,)), ((), ())),
                             preferred_element_type=jnp.float32) + erb_ref[...]
    r = jnp.maximum(_ln(r1, erg_ref[...], erbt_ref[...]), 0.0)

    refined_ref[0] = f + r * edge
    edge_ref[0] = edge


@jax.jit
def kernel(xyz, features, geo_w1, geo_b1, geo_g1, geo_bt1, geo_w2, geo_b2,
           ep_w1, ep_b1, ep_g1, ep_bt1, ep_w2, ep_b2, er_w, er_b, er_g, er_bt):
    xyzp = jnp.pad(xyz, ((0, 0), (0, 0), (0, 5)))
    xaT = jnp.swapaxes(xyzp, 1, 2)
    xcat = jnp.concatenate([xyzp, xyzp * xyzp], axis=2)

    gw1 = jnp.pad(geo_w1, ((0, 6), (0, 0)))
    row = lambda v: v.reshape(1, -1)
    epw1a, epw1b = ep_w1[:C], ep_w1[C:]
    epw2r = ep_w2.reshape(1, -1)
    epb2 = ep_b2.reshape(1, 1)

    grid = (B, N // R)
    wspec = lambda a: pl.BlockSpec(a.shape, lambda b, r: (0,) * a.ndim)

    xr_s = pl.BlockSpec((1, R, 8), lambda b, r: (b, r, 0))
    xaT_s = pl.BlockSpec((1, 8, N), lambda b, r: (b, 0, 0))
    xcat_s = pl.BlockSpec((1, N, 16), lambda b, r: (b, 0, 0))
    f_s = pl.BlockSpec((1, R, C), lambda b, r: (b, r, 0))

    args = (xyzp, xaT, xcat, features,
            gw1, row(geo_b1), row(geo_g1), row(geo_bt1), geo_w2, row(geo_b2),
            epw1a, epw1b, row(ep_b1), row(ep_g1), row(ep_bt1),
            epw2r, epb2,
            er_w, row(er_b), row(er_g), row(er_bt))
    in_specs = [xr_s, xaT_s, xcat_s, f_s] + [wspec(a) for a in args[4:]]

    refined, edge = pl.pallas_call(
        _body,
        grid=grid,
        in_specs=in_specs,
        out_specs=[pl.BlockSpec((1, R, C), lambda b, r: (b, r, 0)),
                   pl.BlockSpec((1, R, 1), lambda b, r: (b, r, 0))],
        out_shape=[jax.ShapeDtypeStruct((B, N, C), jnp.float32),
                   jax.ShapeDtypeStruct((B, N, 1), jnp.float32)],
        compiler_params=pltpu.CompilerParams(
            dimension_semantics=("parallel", "parallel")),
    )(*args)
    return refined, edge
